# baseline (speedup 1.0000x reference)
import jax
import jax.numpy as jnp
from jax.experimental import pallas as pl
from jax.experimental.pallas import tpu as pltpu

_EPS = 1e-5


def _fused_kernel(x_ref, w1_ref, adj_ref, s1_ref, sh1_ref, w2_ref,
                  s2_ref, sh2_ref, dw_ref, db_ref,
                  mu_ref, out_ref, support_ref, t_ref):
    phase = pl.program_id(0)
    i = pl.program_id(1)
    nblk = pl.num_programs(1)

    @pl.when((phase == 0) & (i == 0))
    def _():
        support_ref[...] = jnp.dot(x_ref[...], w1_ref[...],
                                   preferred_element_type=jnp.float32)

    @pl.when(phase == 0)
    def _():
        acc = jnp.dot(adj_ref[...], support_ref[...],
                      preferred_element_type=jnp.float32)
        h1 = jnp.maximum(acc * s1_ref[...] + sh1_ref[...], 0.0)
        bm = adj_ref.shape[0]
        t_ref[pl.ds(i * bm, bm), :] = jnp.dot(
            h1, w2_ref[...], preferred_element_type=jnp.float32)

    @pl.when(phase == 1)
    def _():
        acc = jnp.dot(adj_ref[...], t_ref[...],
                      preferred_element_type=jnp.float32)
        mu = acc * s2_ref[...] + sh2_ref[...]
        mu_ref[...] = mu[None]
        out_ref[...] = (jnp.dot(mu, dw_ref[...],
                                preferred_element_type=jnp.float32)
                        + db_ref[...])[None]


def kernel(x, adj, W1, W2, g1, b1, m1, v1, g2, b2, m2, v2, dec_W, dec_b):
    N, F = x.shape
    H1 = W1.shape[1]
    H2 = W2.shape[1]
    C = dec_W.shape[0]

    inv1 = g1 / jnp.sqrt(v1 + _EPS)
    s1 = inv1.reshape(1, H1)
    sh1 = (b1 - m1 * inv1).reshape(1, H1)
    inv2 = g2 / jnp.sqrt(v2 + _EPS)
    s2 = inv2.reshape(1, H2)
    sh2 = (b2 - m2 * inv2).reshape(1, H2)
    dwT = dec_W.T
    db = dec_b.reshape(1, C)

    BM = 400
    grid = (2, N // BM)

    const = lambda p, i: (0, 0)
    mu, out = pl.pallas_call(
        _fused_kernel,
        grid=grid,
        in_specs=[
            pl.BlockSpec((N, F), const),
            pl.BlockSpec((F, H1), const),
            pl.BlockSpec((BM, N), lambda p, i: (i, 0)),
            pl.BlockSpec((1, H1), const),
            pl.BlockSpec((1, H1), const),
            pl.BlockSpec((H1, H2), const),
            pl.BlockSpec((1, H2), const),
            pl.BlockSpec((1, H2), const),
            pl.BlockSpec((H2, C), const),
            pl.BlockSpec((1, C), const),
        ],
        out_specs=[
            pl.BlockSpec((1, BM, H2), lambda p, i: (p, i, 0)),
            pl.BlockSpec((1, BM, C), lambda p, i: (p, i, 0)),
        ],
        out_shape=[
            jax.ShapeDtypeStruct((2, N, H2), jnp.float32),
            jax.ShapeDtypeStruct((2, N, C), jnp.float32),
        ],
        scratch_shapes=[
            pltpu.VMEM((N, H1), jnp.float32),
            pltpu.VMEM((N, H2), jnp.float32),
        ],
    )(x, W1, adj, s1, sh1, W2, s2, sh2, dwT, db)

    mu = mu[1]
    out = out[1]
    return (out, mu, mu)
